# baseline (speedup 1.0000x reference)
import functools

import jax
import jax.numpy as jnp
from jax.experimental import pallas as pl
from jax.experimental.pallas import tpu as pltpu

N_ROWS = 32 * 1024
K_CODES = 1024
D = 64
BLOCK_R = 4096


def _vq_block(x_ref, emb_ref, mean_ref, std_ref, q_ref, idx_ref, loss_ref,
              emb_s, b2h_s, embh_s):
    i = pl.program_id(0)

    @pl.when(i == 0)
    def _():
        emb = mean_ref[...] + std_ref[...] * emb_ref[...]
        emb_s[...] = emb
        b2h_s[...] = 0.5 * jnp.sum(emb * emb, axis=1)[None, :]
        embh_s[...] = emb.astype(jnp.bfloat16)
        loss_ref[...] = jnp.zeros_like(loss_ref)

    x = x_ref[...]
    xg = jax.lax.dot_general(x, emb_s[...], (((1,), (1,)), ((), ())),
                             preferred_element_type=jnp.float32)
    score = b2h_s[...] - xg

    cmin = jnp.min(score, axis=1, keepdims=True)
    lane = jax.lax.broadcasted_iota(jnp.int32, score.shape, 1)
    idx = jnp.min(jnp.where(score <= cmin, lane, K_CODES), axis=1)
    idx_ref[...] = idx

    onehot = (lane == idx[:, None]).astype(jnp.bfloat16)
    q_ref[...] = jax.lax.dot_general(onehot, embh_s[...],
                                     (((1,), (0,)), ((), ())),
                                     preferred_element_type=jnp.float32)

    a2 = jnp.sum(x * x, axis=1, keepdims=True)
    d2min = jnp.maximum(a2 + 2.0 * cmin, 0.0)
    loss_ref[...] += jnp.sum(d2min, axis=(0, 1), keepdims=True)


@functools.partial(jax.jit, static_argnames=())
def kernel(x, embedding, affine_mean, affine_std):
    flat_x = x.reshape(-1, D)
    mean2 = affine_mean.reshape(1, D)
    std2 = affine_std.reshape(1, D)
    grid = (N_ROWS // BLOCK_R,)
    q, idx, loss_sum = pl.pallas_call(
        _vq_block,
        grid=grid,
        in_specs=[
            pl.BlockSpec((BLOCK_R, D), lambda i: (i, 0)),
            pl.BlockSpec((K_CODES, D), lambda i: (0, 0)),
            pl.BlockSpec((1, D), lambda i: (0, 0)),
            pl.BlockSpec((1, D), lambda i: (0, 0)),
        ],
        out_specs=[
            pl.BlockSpec((BLOCK_R, D), lambda i: (i, 0)),
            pl.BlockSpec((BLOCK_R,), lambda i: (i,)),
            pl.BlockSpec((1, 1), lambda i: (0, 0)),
        ],
        out_shape=[
            jax.ShapeDtypeStruct((N_ROWS, D), jnp.float32),
            jax.ShapeDtypeStruct((N_ROWS,), jnp.int32),
            jax.ShapeDtypeStruct((1, 1), jnp.float32),
        ],
        scratch_shapes=[
            pltpu.VMEM((K_CODES, D), jnp.float32),
            pltpu.VMEM((1, K_CODES), jnp.float32),
            pltpu.VMEM((K_CODES, D), jnp.bfloat16),
        ],
    )(flat_x, embedding, mean2, std2)
    vq_loss = 2.0 * loss_sum[0, 0] / (N_ROWS * D)
    return q.reshape(x.shape), vq_loss, idx


# speedup vs baseline: 1.0816x; 1.0816x over previous
import functools

import jax
import jax.numpy as jnp
from jax.experimental import pallas as pl
from jax.experimental.pallas import tpu as pltpu

N_ROWS = 32 * 1024
K_CODES = 1024
D = 64
BLOCK_R = 4096


def _vq_block(x_ref, emb_ref, mean_ref, std_ref, q_ref, idx_ref, loss_ref,
              emb_s, embh_s):
    i = pl.program_id(0)

    @pl.when(i == 0)
    def _():
        emb = mean_ref[...] + std_ref[...] * emb_ref[...]
        b2h = 0.5 * jnp.sum(emb * emb, axis=1, keepdims=True)
        emb_s[...] = jnp.concatenate((-emb, b2h), axis=1)
        embh_s[...] = emb.astype(jnp.bfloat16)
        loss_ref[...] = jnp.zeros_like(loss_ref)

    x = x_ref[...]
    ones = jnp.ones((x.shape[0], 1), jnp.float32)
    x1 = jnp.concatenate((x, ones), axis=1)
    score = jax.lax.dot_general(x1, emb_s[...], (((1,), (1,)), ((), ())),
                                preferred_element_type=jnp.float32)

    idx = jnp.argmin(score, axis=1).astype(jnp.int32)
    idx_ref[...] = idx

    lane = jax.lax.broadcasted_iota(jnp.int32, score.shape, 1)
    onehot = (lane == idx[:, None]).astype(jnp.bfloat16)
    q = jax.lax.dot_general(onehot, embh_s[...], (((1,), (0,)), ((), ())),
                            preferred_element_type=jnp.float32)
    q_ref[...] = q

    diff = q - x
    loss_ref[...] += jnp.sum(diff * diff, axis=(0, 1), keepdims=True)


@functools.partial(jax.jit, static_argnames=())
def kernel(x, embedding, affine_mean, affine_std):
    flat_x = x.reshape(-1, D)
    mean2 = affine_mean.reshape(1, D)
    std2 = affine_std.reshape(1, D)
    grid = (N_ROWS // BLOCK_R,)
    q, idx, loss_sum = pl.pallas_call(
        _vq_block,
        grid=grid,
        in_specs=[
            pl.BlockSpec((BLOCK_R, D), lambda i: (i, 0)),
            pl.BlockSpec((K_CODES, D), lambda i: (0, 0)),
            pl.BlockSpec((1, D), lambda i: (0, 0)),
            pl.BlockSpec((1, D), lambda i: (0, 0)),
        ],
        out_specs=[
            pl.BlockSpec((BLOCK_R, D), lambda i: (i, 0)),
            pl.BlockSpec((BLOCK_R,), lambda i: (i,)),
            pl.BlockSpec((1, 1), lambda i: (0, 0)),
        ],
        out_shape=[
            jax.ShapeDtypeStruct((N_ROWS, D), jnp.float32),
            jax.ShapeDtypeStruct((N_ROWS,), jnp.int32),
            jax.ShapeDtypeStruct((1, 1), jnp.float32),
        ],
        scratch_shapes=[
            pltpu.VMEM((K_CODES, D + 1), jnp.float32),
            pltpu.VMEM((K_CODES, D), jnp.bfloat16),
        ],
    )(flat_x, embedding, mean2, std2)
    vq_loss = 2.0 * loss_sum[0, 0] / (N_ROWS * D)
    return q.reshape(x.shape), vq_loss, idx
